# baseline (speedup 1.0000x reference)
import jax
import jax.numpy as jnp
from jax import lax
from jax.experimental import pallas as pl
from jax.experimental.pallas import tpu as pltpu, tpu_sc as plsc

N = 8192
L = N * (N - 1) // 2
NC = 2
NS = 16
NW = NC * NS
G = 8
C = 2048
KT = N // C
NT = (N // G) * KT
MT = NT // NW
NQ = C // 16
SROW = C + 32
A8MAX = L - C - 16


def _body(comp, out, stage0, stage1, ob0, ob1, zb, isem0, isem1, osem0, osem1):
    wid = lax.axis_index("s") * NC + lax.axis_index("c")
    iota = lax.iota(jnp.int32, 16)
    zeros16 = jnp.zeros((16,), jnp.float32)

    for r in range(G):
        @plsc.parallel_loop(0, NQ, unroll=4)
        def _z(q, _r=r):
            zb[_r, pl.ds(q * 16, 16)] = zeros16

    def tile_of(m):
        idx = m * NW + wid
        g = idx & (N // G - 1)
        k = idx >> 10
        return g, k

    def is_zero_tile(g, k):
        return (k + 1) * C <= G * g

    def row_src(g, k, r):
        i = G * g + r
        b = i * (N - 1) - (i * (i - 1)) // 2 - i - 1 + k * C
        a8 = jnp.clip((b // 8) * 8, 0, A8MAX)
        return i, a8, b - a8

    def issue_in(m, stg, sem):
        g, k = tile_of(m)

        @pl.when(jnp.logical_not(is_zero_tile(g, k)))
        def _():
            for r in range(G):
                _, a8, _2 = row_src(g, k, r)
                pltpu.async_copy(
                    comp.at[pl.ds(pl.multiple_of(a8, 8), C + 16)],
                    stg.at[pl.ds(r * SROW + 8, C + 16)],
                    sem,
                )

    def wait_in(m, stg, sem):
        pltpu.make_async_copy(
            comp.at[pl.ds(0, G * (C + 16))],
            stg.at[pl.ds(0, G * (C + 16))],
            sem,
        ).wait()

    def wait_out(ob, osem):
        pltpu.make_async_copy(ob, out.at[0, :, pl.ds(0, C)], osem).wait()

    def process(m, stg, ob, isem, osem, not_first, not_last):
        g, k = tile_of(m)
        zero = is_zero_tile(g, k)
        data = jnp.logical_not(zero)
        full = k * C >= G * g + G

        @pl.when(data)
        def _():
            wait_in(m, stg, isem)

        @pl.when(not_first)
        def _():
            wait_out(ob, osem)

        @pl.when(data)
        def _():
            for r in range(G):
                i, _2, rr = row_src(g, k, r)
                colbase = k * C + iota

                @plsc.parallel_loop(0, NQ, unroll=4)
                def _shm(q, _r=r, _rr=rr, _i=i, _cb=colbase):
                    v = stg[pl.ds(_r * SROW + 8 + _rr + q * 16, 16)]
                    col = _cb + q * 16
                    ob[_r, pl.ds(q * 16, 16)] = jnp.where(col <= _i, 0.0, v)

        dst = out.at[g, :, pl.ds(pl.multiple_of(k * C, 128), C)]

        @pl.when(data)
        def _():
            pltpu.async_copy(ob, dst, osem)

        @pl.when(zero)
        def _():
            pltpu.async_copy(zb, dst, osem)

        @pl.when(not_last)
        def _():
            issue_in(m + 2, stg, isem)

    issue_in(0, stage0, isem0)
    issue_in(1, stage1, isem1)

    def pair_body(pp, carry):
        m0 = 2 * pp
        process(m0, stage0, ob0, isem0, osem0, pp > 0, pp < MT // 2 - 1)
        process(m0 + 1, stage1, ob1, isem1, osem1, pp > 0, pp < MT // 2 - 1)
        return carry

    lax.fori_loop(0, MT // 2, pair_body, 0)
    wait_out(ob0, osem0)
    wait_out(ob1, osem1)


def kernel(compressed_matrix):
    mesh = plsc.VectorSubcoreMesh(
        core_axis_name="c", subcore_axis_name="s", num_cores=NC, num_subcores=NS
    )
    f = pl.kernel(
        _body,
        out_type=jax.ShapeDtypeStruct((N // G, G, N), jnp.float32),
        mesh=mesh,
        scratch_types=[
            pltpu.VMEM((G * SROW,), jnp.float32),
            pltpu.VMEM((G * SROW,), jnp.float32),
            pltpu.VMEM((G, C), jnp.float32),
            pltpu.VMEM((G, C), jnp.float32),
            pltpu.VMEM((G, C), jnp.float32),
            pltpu.SemaphoreType.DMA,
            pltpu.SemaphoreType.DMA,
            pltpu.SemaphoreType.DMA,
            pltpu.SemaphoreType.DMA,
        ],
    )
    return f(compressed_matrix).reshape(N, N)


# speedup vs baseline: 1.0772x; 1.0772x over previous
import jax
import jax.numpy as jnp
from jax import lax
from jax.experimental import pallas as pl
from jax.experimental.pallas import tpu as pltpu, tpu_sc as plsc

N = 8192
L = N * (N - 1) // 2
NC = 2
NS = 16
NW = NC * NS
G = 8
C = 2048
KT = N // C
NT = (N // G) * KT
MT = NT // NW
NQ = C // 16
SROW = C + 32
A8MAX = L - C - 16


def _body(comp, out, stage0, stage1, ob0, ob1, zb, isem0, isem1, osem0, osem1):
    wid = lax.axis_index("s") * NC + lax.axis_index("c")
    iota = lax.iota(jnp.int32, 16)
    zeros16 = jnp.zeros((16,), jnp.float32)

    for r in range(G):
        @plsc.parallel_loop(0, NQ, unroll=8)
        def _z(q, _r=r):
            zb[_r, pl.ds(q * 16, 16)] = zeros16

    def tile_of(m):
        idx = m * NW + wid
        g = idx & (N // G - 1)
        k = idx >> 10
        return g, k

    def is_zero_tile(g, k):
        return (k + 1) * C <= G * g

    def row_src(g, k, r):
        i = G * g + r
        b = i * (N - 1) - (i * (i - 1)) // 2 - i - 1 + k * C
        a8 = jnp.clip((b // 8) * 8, 0, A8MAX)
        return i, a8, b - a8

    def issue_in(m, stg, sem):
        g, k = tile_of(m)

        @pl.when(jnp.logical_not(is_zero_tile(g, k)))
        def _():
            for r in range(G):
                _, a8, _2 = row_src(g, k, r)
                pltpu.async_copy(
                    comp.at[pl.ds(pl.multiple_of(a8, 8), C + 16)],
                    stg.at[pl.ds(r * SROW + 8, C + 16)],
                    sem,
                )

    def wait_in(m, stg, sem):
        pltpu.make_async_copy(
            comp.at[pl.ds(0, G * (C + 16))],
            stg.at[pl.ds(0, G * (C + 16))],
            sem,
        ).wait()

    def wait_out(ob, osem):
        pltpu.make_async_copy(ob, out.at[0, :, pl.ds(0, C)], osem).wait()

    def process(m, stg, ob, isem, osem, not_first, not_last):
        g, k = tile_of(m)
        zero = is_zero_tile(g, k)
        data = jnp.logical_not(zero)
        full = k * C >= G * g + G

        dst = out.at[g, :, pl.ds(pl.multiple_of(k * C, 128), C)]

        @pl.when(not_first)
        def _():
            wait_out(ob, osem)

        @pl.when(data)
        def _():
            wait_in(m, stg, isem)
            for r in range(G):
                i, _2, rr = row_src(g, k, r)
                colbase = k * C + iota

                @plsc.parallel_loop(0, NQ, unroll=8)
                def _shm(q, _r=r, _rr=rr, _i=i, _cb=colbase):
                    v = stg[pl.ds(_r * SROW + 8 + _rr + q * 16, 16)]
                    col = _cb + q * 16
                    ob[_r, pl.ds(q * 16, 16)] = jnp.where(col <= _i, 0.0, v)

            pltpu.async_copy(ob, dst, osem)

        @pl.when(zero)
        def _():
            pltpu.async_copy(zb, dst, osem)

        @pl.when(not_last)
        def _():
            issue_in(m + 2, stg, isem)

    issue_in(0, stage0, isem0)
    issue_in(1, stage1, isem1)

    def pair_body(pp, carry):
        m0 = 2 * pp
        process(m0, stage0, ob0, isem0, osem0, pp > 0, pp < MT // 2 - 1)
        process(m0 + 1, stage1, ob1, isem1, osem1, pp > 0, pp < MT // 2 - 1)
        return carry

    lax.fori_loop(0, MT // 2, pair_body, 0)
    wait_out(ob0, osem0)
    wait_out(ob1, osem1)


def kernel(compressed_matrix):
    mesh = plsc.VectorSubcoreMesh(
        core_axis_name="c", subcore_axis_name="s", num_cores=NC, num_subcores=NS
    )
    f = pl.kernel(
        _body,
        out_type=jax.ShapeDtypeStruct((N // G, G, N), jnp.float32),
        mesh=mesh,
        scratch_types=[
            pltpu.VMEM((G * SROW,), jnp.float32),
            pltpu.VMEM((G * SROW,), jnp.float32),
            pltpu.VMEM((G, C), jnp.float32),
            pltpu.VMEM((G, C), jnp.float32),
            pltpu.VMEM((G, C), jnp.float32),
            pltpu.SemaphoreType.DMA,
            pltpu.SemaphoreType.DMA,
            pltpu.SemaphoreType.DMA,
            pltpu.SemaphoreType.DMA,
        ],
    )
    return f(compressed_matrix).reshape(N, N)
